# baseline (speedup 1.0000x reference)
import jax
import jax.numpy as jnp
from jax import lax
from jax.experimental import pallas as pl
from jax.experimental.pallas import tpu as pltpu
from jax.experimental.pallas import tpu_sc as plsc

B, N, T, C = 2, 10000, 4, 32
H, D, K = 2, 16, 2
E = 160000

NP = 10112
EFULL = E + N
BLKE = 64
NBLK = 84
PER_TILE = NBLK * BLKE
NTILES = 32
EP = NTILES * PER_TILE
ROWS_PT = NP // 16

_f32 = jnp.float32


def _stage1_body(x_ref, fc_ref, al_ref, ar_ref, feat_ref, el_ref, er_ref):
    zpad = None
    for k in range(K):
        fw = fc_ref[k]
        alk = al_ref[k]
        ark = ar_ref[k]
        for b in range(B):
            e_chunks = []
            r_chunks = []
            for t in range(T):
                xbt = x_ref[b, :, t, :]
                f = lax.dot_general(xbt, fw, (((1,), (1,)), ((), ())),
                                    preferred_element_type=_f32)
                feat_ref[k, b, :, pl.ds(t * 32, 32)] = f
                e_chunks.append(jnp.dot(f, alk, preferred_element_type=_f32))
                r_chunks.append(jnp.dot(f, ark, preferred_element_type=_f32))
            if b == 0:
                el0, er0 = e_chunks, r_chunks
            else:
                if zpad is None:
                    zpad = jnp.zeros((e_chunks[0].shape[0], 112), _f32)
                el_ref[k] = jnp.concatenate(el0 + e_chunks + [zpad], axis=-1)
                er_ref[k] = jnp.concatenate(er0 + r_chunks + [zpad], axis=-1)


def _stage1(input_feature, fc_w, al, ar):
    bn = 1000
    grid = (N // bn,)
    return pl.pallas_call(
        _stage1_body,
        grid=grid,
        in_specs=[
            pl.BlockSpec((B, bn, T, C), lambda i: (0, i, 0, 0)),
            pl.BlockSpec((K, H * D, C), lambda i: (0, 0, 0)),
            pl.BlockSpec((K, H * D, H), lambda i: (0, 0, 0)),
            pl.BlockSpec((K, H * D, H), lambda i: (0, 0, 0)),
        ],
        out_specs=[
            pl.BlockSpec((K, B, bn, 128), lambda i: (0, 0, i, 0)),
            pl.BlockSpec((K, bn, 128), lambda i: (0, i, 0)),
            pl.BlockSpec((K, bn, 128), lambda i: (0, i, 0)),
        ],
        out_shape=[
            jax.ShapeDtypeStruct((K, B, N, 128), _f32),
            jax.ShapeDtypeStruct((K, N, 128), _f32),
            jax.ShapeDtypeStruct((K, N, 128), _f32),
        ],
    )(input_feature, fc_w, al, ar)


def _bcast_lane(w, lane):
    idx = jnp.full((16, 1), lane, jnp.int32)
    return lax.gather(
        w, idx,
        lax.GatherDimensionNumbers(offset_dims=(), collapsed_slice_dims=(0,),
                                   start_index_map=(0,)),
        slice_sizes=(1,),
        mode=lax.GatherScatterMode.PROMISE_IN_BOUNDS)


_LOG2E = 1.4426950408889634
_MAGIC = 12582912.0
_C1 = 0.6931471805599453
_C2 = 0.2402265069591007
_C3 = 0.05550410866482158
_C4 = 0.009618129107628477
_C5 = 0.0013333558146428443


def _exp_fast(x):
    z = x * _LOG2E
    t = z + _MAGIC
    n = t - _MAGIC
    r = z - n
    p = _C5
    p = p * r + _C4
    p = p * r + _C3
    p = p * r + _C2
    p = p * r + _C1
    p = p * r + 1.0
    ni = n.astype(jnp.int32)
    scale = lax.bitcast_convert_type((ni + 127) << 23, _f32)
    return p * scale


def _sc_body(src2, dst2, el_h, er_h, feat_h, zeros_h, den_out, num_out,
             table, src_v, dst_v, ga, gb, frow, wm,
             sem_a, sem_b, sem_c):
    cid = lax.axis_index("c")
    sid = lax.axis_index("s")
    wid = cid * 16 + sid
    row0 = sid * ROWS_PT

    def zero_table():
        pltpu.sync_copy(zeros_h.at[pl.ds(row0, ROWS_PT)],
                        table.at[pl.ds(row0, ROWS_PT)])
        plsc.subcore_barrier()

    def flush(out_ref):
        plsc.subcore_barrier()
        pltpu.sync_copy(table.at[pl.ds(row0, ROWS_PT)],
                        out_ref.at[pl.ds(row0, ROWS_PT)])
        plsc.subcore_barrier()

    for k in range(K):
        def zrow(i, carry):
            for q in range(1, 8):
                wm[i, pl.ds(q * 16, 16)] = jnp.zeros((16,), _f32)
            return carry
        lax.fori_loop(0, BLKE, zrow, 0)
        zero_table()

        def dblk(j, carry):
            pltpu.sync_copy(src2.at[k, wid, j], src_v)
            pltpu.sync_copy(dst2.at[k, wid, j], dst_v)
            ce = pltpu.async_copy(el_h.at[k].at[src_v], ga, sem_a)
            cr = pltpu.async_copy(er_h.at[k].at[dst_v], gb, sem_b)
            ce.wait()
            cr.wait()

            def edge(i, c2):
                e = ga[i, pl.ds(0, 16)] + gb[i, pl.ds(0, 16)]
                wm[i, pl.ds(0, 16)] = jnp.exp(jnp.maximum(e, 0.2 * e))
                return c2
            lax.fori_loop(0, BLKE, edge, 0, unroll=4)
            pltpu.sync_copy(wm, table.at[dst_v], add=True)
            return carry

        lax.fori_loop(0, NBLK, dblk, 0)
        flush(den_out.at[k, cid])

        for b in range(B):
            zero_table()

            def mblk(j, carry):
                pltpu.sync_copy(src2.at[k, wid, j], src_v)
                pltpu.sync_copy(dst2.at[k, wid, j], dst_v)
                ce = pltpu.async_copy(el_h.at[k].at[src_v], ga, sem_a)
                cr = pltpu.async_copy(er_h.at[k].at[dst_v], gb, sem_b)
                cf = pltpu.async_copy(feat_h.at[k, b].at[src_v], frow, sem_c)
                ce.wait()
                cr.wait()
                cf.wait()

                def edge(i, c2):
                    e = ga[i, pl.ds(0, 16)] + gb[i, pl.ds(0, 16)]
                    w = jnp.exp(jnp.maximum(e, 0.2 * e))
                    for th in range(T * H):
                        sl = pl.ds(th * 16, 16)
                        wb = _bcast_lane(w, b * 8 + th)
                        wm[i, sl] = frow[i, sl] * wb
                    return c2
                lax.fori_loop(0, BLKE, edge, 0, unroll=4)
                pltpu.sync_copy(wm, table.at[dst_v], add=True)
                return carry

            lax.fori_loop(0, NBLK, mblk, 0)
            flush(num_out.at[k, b, cid])


def _stage2(src2, dst2, el, er_pad, feat, zeros_h):
    mesh = plsc.VectorSubcoreMesh(core_axis_name="c", subcore_axis_name="s",
                                  num_cores=2, num_subcores=16)
    kern = pl.kernel(
        _sc_body,
        out_type=[
            jax.ShapeDtypeStruct((K, 2, NP, 128), _f32),
            jax.ShapeDtypeStruct((K, B, 2, NP, 128), _f32),
        ],
        mesh=mesh,
        scratch_types=[
            pltpu.VMEM_SHARED((NP, 128), _f32),
            pltpu.VMEM((BLKE,), jnp.int32),
            pltpu.VMEM((BLKE,), jnp.int32),
            pltpu.VMEM((BLKE, 128), _f32),
            pltpu.VMEM((BLKE, 128), _f32),
            pltpu.VMEM((BLKE, 128), _f32),
            pltpu.VMEM((BLKE, 128), _f32),
            pltpu.SemaphoreType.DMA,
            pltpu.SemaphoreType.DMA,
            pltpu.SemaphoreType.DMA,
        ],
    )
    return kern(src2, dst2, el, er_pad, feat, zeros_h)


def _stage3_body(x_ref, den_ref, num_ref, kw_ref, bias_ref, mw_ref, mb_ref,
                 rw_ref, out_ref):
    dens = [den_ref[k, 0, :, pl.ds(0, 16)] + den_ref[k, 1, :, pl.ds(0, 16)]
            for k in range(K)]
    for b in range(B):
        res = [None] * (T * H)
        for k in range(K):
            kw = kw_ref[k, 0]
            den = dens[k]
            for t in range(T):
                for hh in range(H):
                    j = t * H + hh
                    col = b * 8 + j
                    num = (num_ref[k, b, 0, :, pl.ds(j * 16, 16)]
                           + num_ref[k, b, 1, :, pl.ds(j * 16, 16)])
                    d = den[:, col][:, None]
                    rst = num / d + bias_ref[k, pl.ds(hh * 16, 16)][None, :]
                    contrib = kw * rst
                    res[j] = contrib if res[j] is None else res[j] + contrib
        for t in range(T):
            r_t = jnp.concatenate([res[t * H + hh] for hh in range(H)],
                                  axis=-1)
            merged = lax.dot_general(r_t, mw_ref[...],
                                     (((1,), (1,)), ((), ())),
                                     preferred_element_type=_f32)
            merged = merged + mb_ref[0, :][None, :]
            resid = lax.dot_general(x_ref[b, :, t, :], rw_ref[...],
                                    (((1,), (1,)), ((), ())),
                                    preferred_element_type=_f32)
            out_ref[b, :, t, :] = jnp.maximum(merged, 0.01 * merged) + resid


def _stage3(input_feature, den, num, kernel_weight, gat_bias, merge_w,
            merge_b2, res_w):
    bn = 400
    grid = (N // bn,)
    return pl.pallas_call(
        _stage3_body,
        grid=grid,
        in_specs=[
            pl.BlockSpec((B, bn, T, C), lambda i: (0, i, 0, 0)),
            pl.BlockSpec((K, 2, bn, 128), lambda i: (0, 0, i, 0)),
            pl.BlockSpec((K, B, 2, bn, 128), lambda i: (0, 0, 0, i, 0)),
            pl.BlockSpec((K, 1), lambda i: (0, 0)),
            pl.BlockSpec((K, H * D), lambda i: (0, 0)),
            pl.BlockSpec((D, H * D), lambda i: (0, 0)),
            pl.BlockSpec((1, D), lambda i: (0, 0)),
            pl.BlockSpec((D, C), lambda i: (0, 0)),
        ],
        out_specs=pl.BlockSpec((B, bn, T, D), lambda i: (0, i, 0, 0)),
        out_shape=jax.ShapeDtypeStruct((B, N, T, D), _f32),
    )(input_feature, den, num, kernel_weight, gat_bias, merge_w, merge_b2,
      res_w)


def kernel(input_feature, kd_graph, fc_w, attn_l, attn_r, gat_bias,
           kernel_weight, merge_w, merge_b, res_w):
    rows = jnp.arange(H * D)
    sel = (rows[:, None] // D) == jnp.arange(H)[None, :]
    al = jnp.where(sel[None], attn_l.reshape(K, H * D)[:, :, None], 0.0)
    ar = jnp.where(sel[None], attn_r.reshape(K, H * D)[:, :, None], 0.0)

    feat, el, er = _stage1(input_feature, fc_w, al, ar)
    er_pad = jnp.pad(er, ((0, 0), (0, NP - N), (0, 0)))
    zeros_h = jnp.zeros((NP, 128), _f32)

    self_loop = jnp.arange(N, dtype=jnp.int32)
    npad = EP - EFULL
    pad_src = jnp.zeros((npad,), jnp.int32)
    pad_dst = (N + (jnp.arange(npad) % 16)).astype(jnp.int32)
    src2 = jnp.stack([
        jnp.concatenate([kd_graph[k, 0].astype(jnp.int32), self_loop, pad_src])
        for k in range(K)]).reshape(K, NTILES, NBLK, BLKE)
    dst2 = jnp.stack([
        jnp.concatenate([kd_graph[k, 1].astype(jnp.int32), self_loop, pad_dst])
        for k in range(K)]).reshape(K, NTILES, NBLK, BLKE)

    den, num = _stage2(src2, dst2, el, er_pad, feat, zeros_h)

    out = _stage3(input_feature, den, num,
                  kernel_weight.astype(_f32), gat_bias,
                  merge_w, merge_b.reshape(1, D), res_w)
    return out


# speedup vs baseline: 1.3917x; 1.3917x over previous
import jax
import jax.numpy as jnp
from jax import lax
from jax.experimental import pallas as pl
from jax.experimental.pallas import tpu as pltpu
from jax.experimental.pallas import tpu_sc as plsc

B, N, T, C = 2, 10000, 4, 32
H, D, K = 2, 16, 2
E = 160000

NP = 10112
EFULL = E + N
BLKE = 64
NBLK = 84
PER_TILE = NBLK * BLKE
NTILES = 32
EP = NTILES * PER_TILE
ROWS_PT = NP // 16

_f32 = jnp.float32


def _stage1_body(x_ref, fc_ref, al_ref, ar_ref, feat_ref, el_ref, er_ref):
    zpad = None
    for k in range(K):
        fw = fc_ref[k]
        alk = al_ref[k]
        ark = ar_ref[k]
        for b in range(B):
            e_chunks = []
            r_chunks = []
            for t in range(T):
                xbt = x_ref[b, :, t, :]
                f = lax.dot_general(xbt, fw, (((1,), (1,)), ((), ())),
                                    preferred_element_type=_f32)
                feat_ref[k, b, :, pl.ds(t * 32, 32)] = f
                e_chunks.append(jnp.dot(f, alk, preferred_element_type=_f32))
                r_chunks.append(jnp.dot(f, ark, preferred_element_type=_f32))
            if b == 0:
                el0, er0 = e_chunks, r_chunks
            else:
                if zpad is None:
                    zpad = jnp.zeros((e_chunks[0].shape[0], 112), _f32)
                el_ref[k] = jnp.concatenate(el0 + e_chunks + [zpad], axis=-1)
                er_ref[k] = jnp.concatenate(er0 + r_chunks + [zpad], axis=-1)


def _stage1(input_feature, fc_w, al, ar):
    bn = 1000
    grid = (N // bn,)
    return pl.pallas_call(
        _stage1_body,
        grid=grid,
        in_specs=[
            pl.BlockSpec((B, bn, T, C), lambda i: (0, i, 0, 0)),
            pl.BlockSpec((K, H * D, C), lambda i: (0, 0, 0)),
            pl.BlockSpec((K, H * D, H), lambda i: (0, 0, 0)),
            pl.BlockSpec((K, H * D, H), lambda i: (0, 0, 0)),
        ],
        out_specs=[
            pl.BlockSpec((K, B, bn, 128), lambda i: (0, 0, i, 0)),
            pl.BlockSpec((K, bn, 128), lambda i: (0, i, 0)),
            pl.BlockSpec((K, bn, 128), lambda i: (0, i, 0)),
        ],
        out_shape=[
            jax.ShapeDtypeStruct((K, B, N, 128), _f32),
            jax.ShapeDtypeStruct((K, N, 128), _f32),
            jax.ShapeDtypeStruct((K, N, 128), _f32),
        ],
    )(input_feature, fc_w, al, ar)


def _bcast_lane(w, lane):
    idx = jnp.full((16, 1), lane, jnp.int32)
    return lax.gather(
        w, idx,
        lax.GatherDimensionNumbers(offset_dims=(), collapsed_slice_dims=(0,),
                                   start_index_map=(0,)),
        slice_sizes=(1,),
        mode=lax.GatherScatterMode.PROMISE_IN_BOUNDS)


_LOG2E = 1.4426950408889634
_MAGIC = 12582912.0
_C1 = 0.6931471805599453
_C2 = 0.2402265069591007
_C3 = 0.05550410866482158
_C4 = 0.009618129107628477
_C5 = 0.0013333558146428443


def _exp_fast(x):
    z = x * _LOG2E
    t = z + _MAGIC
    n = t - _MAGIC
    r = z - n
    p = _C5
    p = p * r + _C4
    p = p * r + _C3
    p = p * r + _C2
    p = p * r + _C1
    p = p * r + 1.0
    ni = n.astype(jnp.int32)
    scale = lax.bitcast_convert_type((ni + 127) << 23, _f32)
    return p * scale


def _sc_body(src2, dst2, el_h, er_h, feat_h, zeros_h, den_out, num_out,
             table, src_v, dst_v, ga, gb, frow, wm,
             sem_a, sem_b, sem_c):
    cid = lax.axis_index("c")
    sid = lax.axis_index("s")
    wid = cid * 16 + sid
    row0 = sid * ROWS_PT

    def zero_table():
        pltpu.sync_copy(zeros_h.at[pl.ds(row0, ROWS_PT)],
                        table.at[pl.ds(row0, ROWS_PT)])
        plsc.subcore_barrier()

    def flush(out_ref):
        plsc.subcore_barrier()
        pltpu.sync_copy(table.at[pl.ds(row0, ROWS_PT)],
                        out_ref.at[pl.ds(row0, ROWS_PT)])
        plsc.subcore_barrier()

    for k in range(K):
        def zrow(i, carry):
            for q in range(1, 8):
                wm[i, pl.ds(q * 16, 16)] = jnp.zeros((16,), _f32)
            return carry
        lax.fori_loop(0, BLKE, zrow, 0)
        zero_table()

        def dblk(j, carry):
            pltpu.sync_copy(src2.at[k, wid, j], src_v)
            pltpu.sync_copy(dst2.at[k, wid, j], dst_v)
            ce = pltpu.async_copy(el_h.at[k].at[src_v], ga, sem_a)
            cr = pltpu.async_copy(er_h.at[k].at[dst_v], gb, sem_b)
            ce.wait()
            cr.wait()

            def edge(i, c2):
                e = ga[i, pl.ds(0, 16)] + gb[i, pl.ds(0, 16)]
                wm[i, pl.ds(0, 16)] = jnp.exp(jnp.maximum(e, 0.2 * e))
                return c2
            lax.fori_loop(0, BLKE, edge, 0)
            pltpu.sync_copy(wm, table.at[dst_v], add=True)
            return carry

        lax.fori_loop(0, NBLK, dblk, 0)
        flush(den_out.at[k, cid])

        for b in range(B):
            zero_table()

            def mblk(j, carry):
                pltpu.sync_copy(src2.at[k, wid, j], src_v)
                pltpu.sync_copy(dst2.at[k, wid, j], dst_v)
                ce = pltpu.async_copy(el_h.at[k].at[src_v], ga, sem_a)
                cr = pltpu.async_copy(er_h.at[k].at[dst_v], gb, sem_b)
                cf = pltpu.async_copy(feat_h.at[k, b].at[src_v], frow, sem_c)
                ce.wait()
                cr.wait()
                cf.wait()

                def edge(i, c2):
                    e = ga[i, pl.ds(0, 16)] + gb[i, pl.ds(0, 16)]
                    w = jnp.exp(jnp.maximum(e, 0.2 * e))
                    for th in range(T * H):
                        sl = pl.ds(th * 16, 16)
                        wb = _bcast_lane(w, b * 8 + th)
                        wm[i, sl] = frow[i, sl] * wb
                    return c2
                lax.fori_loop(0, BLKE, edge, 0)
                pltpu.sync_copy(wm, table.at[dst_v], add=True)
                return carry

            lax.fori_loop(0, NBLK, mblk, 0)
            flush(num_out.at[k, b, cid])


def _stage2(src2, dst2, el, er_pad, feat, zeros_h):
    mesh = plsc.VectorSubcoreMesh(core_axis_name="c", subcore_axis_name="s",
                                  num_cores=2, num_subcores=16)
    kern = pl.kernel(
        _sc_body,
        out_type=[
            jax.ShapeDtypeStruct((K, 2, NP, 128), _f32),
            jax.ShapeDtypeStruct((K, B, 2, NP, 128), _f32),
        ],
        mesh=mesh,
        scratch_types=[
            pltpu.VMEM_SHARED((NP, 128), _f32),
            pltpu.VMEM((BLKE,), jnp.int32),
            pltpu.VMEM((BLKE,), jnp.int32),
            pltpu.VMEM((BLKE, 128), _f32),
            pltpu.VMEM((BLKE, 128), _f32),
            pltpu.VMEM((BLKE, 128), _f32),
            pltpu.VMEM((BLKE, 128), _f32),
            pltpu.SemaphoreType.DMA,
            pltpu.SemaphoreType.DMA,
            pltpu.SemaphoreType.DMA,
        ],
    )
    return kern(src2, dst2, el, er_pad, feat, zeros_h)


def _stage3_body(x_ref, den_ref, num_ref, kw_ref, bias_ref, mw_ref, mb_ref,
                 rw_ref, out_ref):
    dens = [den_ref[k, 0, :, pl.ds(0, 16)] + den_ref[k, 1, :, pl.ds(0, 16)]
            for k in range(K)]
    for b in range(B):
        res = [None] * (T * H)
        for k in range(K):
            kw = kw_ref[k, 0]
            den = dens[k]
            for t in range(T):
                for hh in range(H):
                    j = t * H + hh
                    col = b * 8 + j
                    num = (num_ref[k, b, 0, :, pl.ds(j * 16, 16)]
                           + num_ref[k, b, 1, :, pl.ds(j * 16, 16)])
                    d = den[:, col][:, None]
                    rst = num / d + bias_ref[k, pl.ds(hh * 16, 16)][None, :]
                    contrib = kw * rst
                    res[j] = contrib if res[j] is None else res[j] + contrib
        for t in range(T):
            r_t = jnp.concatenate([res[t * H + hh] for hh in range(H)],
                                  axis=-1)
            merged = lax.dot_general(r_t, mw_ref[...],
                                     (((1,), (1,)), ((), ())),
                                     preferred_element_type=_f32)
            merged = merged + mb_ref[0, :][None, :]
            resid = lax.dot_general(x_ref[b, :, t, :], rw_ref[...],
                                    (((1,), (1,)), ((), ())),
                                    preferred_element_type=_f32)
            out_ref[b, :, t, :] = jnp.maximum(merged, 0.01 * merged) + resid


def _stage3(input_feature, den, num, kernel_weight, gat_bias, merge_w,
            merge_b2, res_w):
    bn = 400
    grid = (N // bn,)
    return pl.pallas_call(
        _stage3_body,
        grid=grid,
        in_specs=[
            pl.BlockSpec((B, bn, T, C), lambda i: (0, i, 0, 0)),
            pl.BlockSpec((K, 2, bn, 128), lambda i: (0, 0, i, 0)),
            pl.BlockSpec((K, B, 2, bn, 128), lambda i: (0, 0, 0, i, 0)),
            pl.BlockSpec((K, 1), lambda i: (0, 0)),
            pl.BlockSpec((K, H * D), lambda i: (0, 0)),
            pl.BlockSpec((D, H * D), lambda i: (0, 0)),
            pl.BlockSpec((1, D), lambda i: (0, 0)),
            pl.BlockSpec((D, C), lambda i: (0, 0)),
        ],
        out_specs=pl.BlockSpec((B, bn, T, D), lambda i: (0, i, 0, 0)),
        out_shape=jax.ShapeDtypeStruct((B, N, T, D), _f32),
    )(input_feature, den, num, kernel_weight, gat_bias, merge_w, merge_b2,
      res_w)


def kernel(input_feature, kd_graph, fc_w, attn_l, attn_r, gat_bias,
           kernel_weight, merge_w, merge_b, res_w):
    rows = jnp.arange(H * D)
    sel = (rows[:, None] // D) == jnp.arange(H)[None, :]
    al = jnp.where(sel[None], attn_l.reshape(K, H * D)[:, :, None], 0.0)
    ar = jnp.where(sel[None], attn_r.reshape(K, H * D)[:, :, None], 0.0)

    feat, el, er = _stage1(input_feature, fc_w, al, ar)
    er_pad = jnp.pad(er, ((0, 0), (0, NP - N), (0, 0)))
    zeros_h = jnp.zeros((NP, 128), _f32)

    self_loop = jnp.arange(N, dtype=jnp.int32)
    npad = EP - EFULL
    pad_src = jnp.zeros((npad,), jnp.int32)
    pad_dst = (N + (jnp.arange(npad) % 16)).astype(jnp.int32)
    src2 = jnp.stack([
        jnp.concatenate([kd_graph[k, 0].astype(jnp.int32), self_loop, pad_src])
        for k in range(K)]).reshape(K, NTILES, NBLK, BLKE)
    dst2 = jnp.stack([
        jnp.concatenate([kd_graph[k, 1].astype(jnp.int32), self_loop, pad_dst])
        for k in range(K)]).reshape(K, NTILES, NBLK, BLKE)

    den, num = _stage2(src2, dst2, el, er_pad, feat, zeros_h)

    out = _stage3(input_feature, den, num,
                  kernel_weight.astype(_f32), gat_bias,
                  merge_w, merge_b.reshape(1, D), res_w)
    return out


# speedup vs baseline: 1.4075x; 1.0114x over previous
import jax
import jax.numpy as jnp
from jax import lax
from jax.experimental import pallas as pl
from jax.experimental.pallas import tpu as pltpu
from jax.experimental.pallas import tpu_sc as plsc

B, N, T, C = 2, 10000, 4, 32
H, D, K = 2, 16, 2
E = 160000

NP = 10112
EFULL = E + N
BLKE = 64
NBLK = 84
PER_TILE = NBLK * BLKE
NTILES = 32
EP = NTILES * PER_TILE
ROWS_PT = NP // 16

_f32 = jnp.float32


def _stage1_body(x_ref, fc_ref, al_ref, ar_ref, feat_ref, el_ref, er_ref):
    zpad = None
    for k in range(K):
        fw = fc_ref[k]
        alk = al_ref[k]
        ark = ar_ref[k]
        for b in range(B):
            e_chunks = []
            r_chunks = []
            for t in range(T):
                xbt = x_ref[b, :, t, :]
                f = lax.dot_general(xbt, fw, (((1,), (1,)), ((), ())),
                                    preferred_element_type=_f32)
                feat_ref[k, b, :, pl.ds(t * 32, 32)] = f
                e_chunks.append(jnp.dot(f, alk, preferred_element_type=_f32))
                r_chunks.append(jnp.dot(f, ark, preferred_element_type=_f32))
            if b == 0:
                el0, er0 = e_chunks, r_chunks
            else:
                if zpad is None:
                    zpad = jnp.zeros((e_chunks[0].shape[0], 112), _f32)
                el_ref[k] = jnp.concatenate(el0 + e_chunks + [zpad], axis=-1)
                er_ref[k] = jnp.concatenate(er0 + r_chunks + [zpad], axis=-1)


def _stage1(input_feature, fc_w, al, ar):
    bn = 1000
    grid = (N // bn,)
    return pl.pallas_call(
        _stage1_body,
        grid=grid,
        in_specs=[
            pl.BlockSpec((B, bn, T, C), lambda i: (0, i, 0, 0)),
            pl.BlockSpec((K, H * D, C), lambda i: (0, 0, 0)),
            pl.BlockSpec((K, H * D, H), lambda i: (0, 0, 0)),
            pl.BlockSpec((K, H * D, H), lambda i: (0, 0, 0)),
        ],
        out_specs=[
            pl.BlockSpec((K, B, bn, 128), lambda i: (0, 0, i, 0)),
            pl.BlockSpec((K, bn, 128), lambda i: (0, i, 0)),
            pl.BlockSpec((K, bn, 128), lambda i: (0, i, 0)),
        ],
        out_shape=[
            jax.ShapeDtypeStruct((K, B, N, 128), _f32),
            jax.ShapeDtypeStruct((K, N, 128), _f32),
            jax.ShapeDtypeStruct((K, N, 128), _f32),
        ],
    )(input_feature, fc_w, al, ar)


def _bcast_lane(w, lane):
    idx = jnp.full((16, 1), lane, jnp.int32)
    return lax.gather(
        w, idx,
        lax.GatherDimensionNumbers(offset_dims=(), collapsed_slice_dims=(0,),
                                   start_index_map=(0,)),
        slice_sizes=(1,),
        mode=lax.GatherScatterMode.PROMISE_IN_BOUNDS)


_LOG2E = 1.4426950408889634
_MAGIC = 12582912.0
_C1 = 0.6931471805599453
_C2 = 0.2402265069591007
_C3 = 0.05550410866482158
_C4 = 0.009618129107628477
_C5 = 0.0013333558146428443


def _exp_fast(x):
    z = x * _LOG2E
    t = z + _MAGIC
    n = t - _MAGIC
    r = z - n
    p = _C5
    p = p * r + _C4
    p = p * r + _C3
    p = p * r + _C2
    p = p * r + _C1
    p = p * r + 1.0
    ni = n.astype(jnp.int32)
    scale = lax.bitcast_convert_type((ni + 127) << 23, _f32)
    return p * scale


def _sc_body(src2, dst2, el_h, er_h, feat_h, zeros_h, wside,
             den_out, num_out,
             table, src_v, dst_v, ga, gb, frow, wm, wlin,
             sem_a, sem_b, sem_c):
    cid = lax.axis_index("c")
    sid = lax.axis_index("s")
    wid = cid * 16 + sid
    row0 = sid * ROWS_PT
    ebase = wid * PER_TILE

    def zero_table():
        pltpu.sync_copy(zeros_h.at[pl.ds(row0, ROWS_PT)],
                        table.at[pl.ds(row0, ROWS_PT)])
        plsc.subcore_barrier()

    def flush(out_ref):
        plsc.subcore_barrier()
        pltpu.sync_copy(table.at[pl.ds(row0, ROWS_PT)],
                        out_ref.at[pl.ds(row0, ROWS_PT)])
        plsc.subcore_barrier()

    for k in range(K):
        def zrow(i, carry):
            for q in range(1, 8):
                wm[i, pl.ds(q * 16, 16)] = jnp.zeros((16,), _f32)
            return carry
        lax.fori_loop(0, BLKE, zrow, 0)
        zero_table()

        def dblk(j, carry):
            pltpu.sync_copy(src2.at[k, wid, j], src_v)
            pltpu.sync_copy(dst2.at[k, wid, j], dst_v)
            ce = pltpu.async_copy(el_h.at[k].at[src_v], ga, sem_a)
            cr = pltpu.async_copy(er_h.at[k].at[dst_v], gb, sem_b)
            ce.wait()
            cr.wait()

            def edge(i, c2):
                e = ga[i, pl.ds(0, 16)] + gb[i, pl.ds(0, 16)]
                w = jnp.exp(jnp.maximum(e, 0.2 * e))
                wm[i, pl.ds(0, 16)] = w
                wlin[pl.ds(i * 16, 16)] = w
                return c2
            lax.fori_loop(0, BLKE, edge, 0)
            cs = pltpu.async_copy(
                wlin, wside.at[pl.ds((k * EP + ebase + j * BLKE) * 16,
                                     BLKE * 16)], sem_c)
            pltpu.sync_copy(wm, table.at[dst_v], add=True)
            cs.wait()
            return carry

        lax.fori_loop(0, NBLK, dblk, 0)
        flush(den_out.at[k, cid])

        for b in range(B):
            zero_table()

            def mblk(j, carry):
                pltpu.sync_copy(dst2.at[k, wid, j], dst_v)
                pltpu.sync_copy(src2.at[k, wid, j], src_v)
                cw = pltpu.async_copy(
                    wside.at[pl.ds((k * EP + ebase + j * BLKE) * 16,
                                   BLKE * 16)], wlin, sem_a)
                cf = pltpu.async_copy(feat_h.at[k, b].at[src_v], frow, sem_b)
                cw.wait()
                cf.wait()

                def edge(i, c2):
                    w = wlin[pl.ds(i * 16, 16)]
                    for th in range(T * H):
                        sl = pl.ds(th * 16, 16)
                        wb = _bcast_lane(w, b * 8 + th)
                        wm[i, sl] = frow[i, sl] * wb
                    return c2
                lax.fori_loop(0, BLKE, edge, 0)
                pltpu.sync_copy(wm, table.at[dst_v], add=True)
                return carry

            lax.fori_loop(0, NBLK, mblk, 0)
            flush(num_out.at[k, b, cid])


def _stage2(src2, dst2, el, er_pad, feat, zeros_h):
    mesh = plsc.VectorSubcoreMesh(core_axis_name="c", subcore_axis_name="s",
                                  num_cores=2, num_subcores=16)
    kern = pl.kernel(
        _sc_body,
        out_type=[
            jax.ShapeDtypeStruct((K * EP * 16,), _f32),
            jax.ShapeDtypeStruct((K, 2, NP, 128), _f32),
            jax.ShapeDtypeStruct((K, B, 2, NP, 128), _f32),
        ],
        mesh=mesh,
        scratch_types=[
            pltpu.VMEM_SHARED((NP, 128), _f32),
            pltpu.VMEM((BLKE,), jnp.int32),
            pltpu.VMEM((BLKE,), jnp.int32),
            pltpu.VMEM((BLKE, 128), _f32),
            pltpu.VMEM((BLKE, 128), _f32),
            pltpu.VMEM((BLKE, 128), _f32),
            pltpu.VMEM((BLKE, 128), _f32),
            pltpu.VMEM((BLKE * 16,), _f32),
            pltpu.SemaphoreType.DMA,
            pltpu.SemaphoreType.DMA,
            pltpu.SemaphoreType.DMA,
        ],
    )
    _, den, num = kern(src2, dst2, el, er_pad, feat, zeros_h)
    return den, num


def _stage3_body(x_ref, den_ref, num_ref, kw_ref, bias_ref, mw_ref, mb_ref,
                 rw_ref, out_ref):
    dens = [den_ref[k, 0, :, pl.ds(0, 16)] + den_ref[k, 1, :, pl.ds(0, 16)]
            for k in range(K)]
    for b in range(B):
        res = [None] * (T * H)
        for k in range(K):
            kw = kw_ref[k, 0]
            den = dens[k]
            for t in range(T):
                for hh in range(H):
                    j = t * H + hh
                    col = b * 8 + j
                    num = (num_ref[k, b, 0, :, pl.ds(j * 16, 16)]
                           + num_ref[k, b, 1, :, pl.ds(j * 16, 16)])
                    d = den[:, col][:, None]
                    rst = num / d + bias_ref[k, pl.ds(hh * 16, 16)][None, :]
                    contrib = kw * rst
                    res[j] = contrib if res[j] is None else res[j] + contrib
        for t in range(T):
            r_t = jnp.concatenate([res[t * H + hh] for hh in range(H)],
                                  axis=-1)
            merged = lax.dot_general(r_t, mw_ref[...],
                                     (((1,), (1,)), ((), ())),
                                     preferred_element_type=_f32)
            merged = merged + mb_ref[0, :][None, :]
            resid = lax.dot_general(x_ref[b, :, t, :], rw_ref[...],
                                    (((1,), (1,)), ((), ())),
                                    preferred_element_type=_f32)
            out_ref[b, :, t, :] = jnp.maximum(merged, 0.01 * merged) + resid


def _stage3(input_feature, den, num, kernel_weight, gat_bias, merge_w,
            merge_b2, res_w):
    bn = 400
    grid = (N // bn,)
    return pl.pallas_call(
        _stage3_body,
        grid=grid,
        in_specs=[
            pl.BlockSpec((B, bn, T, C), lambda i: (0, i, 0, 0)),
            pl.BlockSpec((K, 2, bn, 128), lambda i: (0, 0, i, 0)),
            pl.BlockSpec((K, B, 2, bn, 128), lambda i: (0, 0, 0, i, 0)),
            pl.BlockSpec((K, 1), lambda i: (0, 0)),
            pl.BlockSpec((K, H * D), lambda i: (0, 0)),
            pl.BlockSpec((D, H * D), lambda i: (0, 0)),
            pl.BlockSpec((1, D), lambda i: (0, 0)),
            pl.BlockSpec((D, C), lambda i: (0, 0)),
        ],
        out_specs=pl.BlockSpec((B, bn, T, D), lambda i: (0, i, 0, 0)),
        out_shape=jax.ShapeDtypeStruct((B, N, T, D), _f32),
    )(input_feature, den, num, kernel_weight, gat_bias, merge_w, merge_b2,
      res_w)


def kernel(input_feature, kd_graph, fc_w, attn_l, attn_r, gat_bias,
           kernel_weight, merge_w, merge_b, res_w):
    rows = jnp.arange(H * D)
    sel = (rows[:, None] // D) == jnp.arange(H)[None, :]
    al = jnp.where(sel[None], attn_l.reshape(K, H * D)[:, :, None], 0.0)
    ar = jnp.where(sel[None], attn_r.reshape(K, H * D)[:, :, None], 0.0)

    feat, el, er = _stage1(input_feature, fc_w, al, ar)
    er_pad = jnp.pad(er, ((0, 0), (0, NP - N), (0, 0)))
    zeros_h = jnp.zeros((NP, 128), _f32)

    self_loop = jnp.arange(N, dtype=jnp.int32)
    npad = EP - EFULL
    pad_src = jnp.zeros((npad,), jnp.int32)
    pad_dst = (N + (jnp.arange(npad) % 16)).astype(jnp.int32)
    src2 = jnp.stack([
        jnp.concatenate([kd_graph[k, 0].astype(jnp.int32), self_loop, pad_src])
        for k in range(K)]).reshape(K, NTILES, NBLK, BLKE)
    dst2 = jnp.stack([
        jnp.concatenate([kd_graph[k, 1].astype(jnp.int32), self_loop, pad_dst])
        for k in range(K)]).reshape(K, NTILES, NBLK, BLKE)

    den, num = _stage2(src2, dst2, el, er_pad, feat, zeros_h)

    out = _stage3(input_feature, den, num,
                  kernel_weight.astype(_f32), gat_bias,
                  merge_w, merge_b.reshape(1, D), res_w)
    return out
